# baseline (speedup 1.0000x reference)
import jax
import jax.numpy as jnp
from jax import lax
from jax.experimental import pallas as pl
from jax.experimental.pallas import tpu as pltpu
from jax.experimental.pallas import tpu_sc as plsc

WORD_DIM = 64
INPUT_DIM = 128
NW = 32
CHUNK = 400
CB = 32768


def _tc_project(tblk_ref, w_ref, b_ref, out_ref):
    acc = lax.dot_general(
        tblk_ref[...], w_ref[...],
        dimension_numbers=(((0,), (0,)), ((), ())),
        preferred_element_type=jnp.float32,
    )
    out_ref[...] = jnp.tanh(acc + b_ref[...])


def _sc_gather(idx_hbm, ptable_hbm, out_hbm,
               idx_all, rows0, rows1, sem0, sem1):
    wid = lax.axis_index("s") * 2 + lax.axis_index("c")
    total = out_hbm.shape[0]
    per_w = total // NW
    n_it = per_w // CHUNK
    base = wid * per_w
    rows = (rows0, rows1)
    sem = (sem0, sem1)

    pltpu.sync_copy(idx_hbm.at[pl.ds(base, per_w)], idx_all)

    def issue(i, p):
        idx = idx_all.at[pl.ds(i * CHUNK, CHUNK)]
        pltpu.async_copy(ptable_hbm.at[idx], rows[p], sem[p])

    issue(0, 0)

    def body(j, carry):
        for p in range(2):
            i = 2 * j + p

            @pl.when(i + 1 < n_it)
            def _():
                issue(i + 1, 1 - p)

            idx = idx_all.at[pl.ds(i * CHUNK, CHUNK)]
            pltpu.make_async_copy(ptable_hbm.at[idx], rows[p], sem[p]).wait()
            pltpu.sync_copy(rows[p], out_hbm.at[pl.ds(base + i * CHUNK, CHUNK)])
        return carry

    lax.fori_loop(0, n_it // 2, body, 0)


def kernel(word_ids, word_table, W, b):
    Bb, L = word_ids.shape
    total = Bb * L
    vocab = word_table.shape[0]
    flat_ids = word_ids.reshape(total).astype(jnp.int32)
    tableT = word_table.T

    grid = pl.cdiv(vocab, CB)
    ptable = pl.pallas_call(
        _tc_project,
        grid=(grid,),
        in_specs=[
            pl.BlockSpec((WORD_DIM, CB), lambda i: (0, i)),
            pl.BlockSpec((WORD_DIM, INPUT_DIM), lambda i: (0, 0)),
            pl.BlockSpec((1, INPUT_DIM), lambda i: (0, 0)),
        ],
        out_specs=pl.BlockSpec((CB, INPUT_DIM), lambda i: (i, 0)),
        out_shape=jax.ShapeDtypeStruct((vocab, INPUT_DIM), jnp.float32),
    )(tableT, W, b.reshape(1, INPUT_DIM))

    mesh = plsc.VectorSubcoreMesh(core_axis_name="c", subcore_axis_name="s")
    gather = pl.kernel(
        _sc_gather,
        mesh=mesh,
        out_type=jax.ShapeDtypeStruct((total, INPUT_DIM), jnp.float32),
        scratch_types=[
            pltpu.VMEM((total // NW,), jnp.int32),
            pltpu.VMEM((CHUNK, INPUT_DIM), jnp.float32),
            pltpu.VMEM((CHUNK, INPUT_DIM), jnp.float32),
            pltpu.SemaphoreType.DMA,
            pltpu.SemaphoreType.DMA,
        ],
    )
    out = gather(flat_ids, ptable)

    def _tc_probe(pt_ref, acc_ref):
        @pl.when(pl.program_id(0) == 0)
        def _():
            acc_ref[...] = jnp.zeros_like(acc_ref)

        acc_ref[...] += jnp.sum(pt_ref[...], axis=0, keepdims=True)

    probe = pl.pallas_call(
        _tc_probe,
        grid=(pl.cdiv(vocab, CB),),
        in_specs=[pl.BlockSpec((CB, INPUT_DIM), lambda i: (i, 0))],
        out_specs=pl.BlockSpec((1, INPUT_DIM), lambda i: (0, 0)),
        out_shape=jax.ShapeDtypeStruct((1, INPUT_DIM), jnp.float32),
    )(ptable)

    out3 = out.reshape(Bb, L, INPUT_DIM)
    out3, _ = lax.optimization_barrier((out3, probe))
    return out3
